# Initial kernel scaffold; baseline (speedup 1.0000x reference)
import functools

import jax
import jax.numpy as jnp
from jax import lax
from jax.experimental import pallas as pl
from jax.experimental.pallas import tpu as pltpu
from jax.experimental.pallas import tpu_sc as plsc

NC = 2
NS = 16
L = 16
NW = NC * NS

F = 5
TPAD = 128


def _encoder_grid(n_idx, chunk):
    per_w = n_idx // NW
    n_chunks = per_w // chunk
    mesh = plsc.VectorSubcoreMesh(core_axis_name="c", subcore_axis_name="s")

    @functools.partial(
        pl.kernel,
        mesh=mesh,
        out_type=jax.ShapeDtypeStruct((n_idx * F,), jnp.float32),
        scratch_types=[
            pltpu.VMEM((chunk,), jnp.int32),
            pltpu.VMEM((chunk * F,), jnp.float32),
            pltpu.VMEM((TPAD,), jnp.float32),
            pltpu.VMEM((TPAD,), jnp.float32),
            pltpu.VMEM((TPAD,), jnp.float32),
            pltpu.VMEM((TPAD,), jnp.float32),
        ],
    )
    def enc(idx_hbm, tab_hbm, mean_hbm, scale_hbm, out_hbm,
            idx_v, out_v, tab_v, mean_v, scale_v, norm_v):
        wid = lax.axis_index("s") * NC + lax.axis_index("c")
        base = wid * per_w

        pltpu.sync_copy(tab_hbm, tab_v)
        pltpu.sync_copy(mean_hbm, mean_v)
        pltpu.sync_copy(scale_hbm, scale_v)
        for k in range(TPAD // L):
            s = pl.ds(k * L, L)
            norm_v[s] = (tab_v[s] - mean_v[s]) / scale_v[s]

        lane = lax.iota(jnp.int32, L)
        qr = []
        for v in range(F):
            j = lane + v * L
            q = lax.shift_right_logical(j * 205, 10)
            r = j - q * 5
            qr.append((q, r))

        def chunk_body(g, _):
            off = base + g * chunk
            pltpu.sync_copy(idx_hbm.at[pl.ds(off, chunk)], idx_v)

            def group_body(i, _):
                i16 = i * L
                for v in range(F):
                    q, r = qr[v]
                    ids = plsc.load_gather(idx_v, [q + i16])
                    vals = plsc.load_gather(norm_v, [ids * F + r])
                    out_v[pl.ds(i * (L * F) + v * L, L)] = vals
                return 0

            lax.fori_loop(0, chunk // L, group_body, 0)
            pltpu.sync_copy(out_v, out_hbm.at[pl.ds(off * F, chunk * F)])
            return 0

        lax.fori_loop(0, n_chunks, chunk_body, 0)

    return enc


def kernel(indices, aa_feature_table, mean_tensor, scale_tensor):
    b, l = indices.shape
    n_idx = b * l
    idx_flat = indices.reshape(n_idx).astype(jnp.int32)

    tab_flat = jnp.pad(aa_feature_table.reshape(-1), (0, TPAD - 22 * F))
    mean_flat = jnp.pad(jnp.tile(mean_tensor, 22), (0, TPAD - 22 * F))
    scale_flat = jnp.pad(jnp.tile(scale_tensor, 22), (0, TPAD - 22 * F),
                         constant_values=1.0)

    enc = _encoder_grid(n_idx, chunk=4096)
    out_flat = enc(idx_flat, tab_flat, mean_flat, scale_flat)
    return out_flat.reshape(b, l, F)


# speedup vs baseline: 5.1457x; 5.1457x over previous
import functools

import jax
import jax.numpy as jnp
from jax import lax
from jax.experimental import pallas as pl
from jax.experimental.pallas import tpu as pltpu
from jax.experimental.pallas import tpu_sc as plsc

NC = 2
NS = 16
L = 16
NW = NC * NS

F = 5
TPAD = 128


def _encoder_grid(n_idx, chunk):
    per_w = n_idx // NW
    n_chunks = per_w // chunk
    mesh = plsc.VectorSubcoreMesh(core_axis_name="c", subcore_axis_name="s")

    @functools.partial(
        pl.kernel,
        mesh=mesh,
        out_type=jax.ShapeDtypeStruct((n_idx * F,), jnp.float32),
        compiler_params=pltpu.CompilerParams(needs_layout_passes=False),
        scratch_types=[
            pltpu.VMEM((chunk,), jnp.int32),
            pltpu.VMEM((chunk * F,), jnp.float32),
            pltpu.VMEM((TPAD,), jnp.float32),
            pltpu.VMEM((TPAD,), jnp.float32),
            pltpu.VMEM((TPAD,), jnp.float32),
            pltpu.VMEM((TPAD,), jnp.float32),
        ],
    )
    def enc(idx_hbm, tab_hbm, mean_hbm, scale_hbm, out_hbm,
            idx_v, out_v, tab_v, mean_v, scale_v, norm_v):
        wid = lax.axis_index("s") * NC + lax.axis_index("c")
        base = wid * per_w

        pltpu.sync_copy(tab_hbm, tab_v)
        pltpu.sync_copy(mean_hbm, mean_v)
        pltpu.sync_copy(scale_hbm, scale_v)
        for k in range(TPAD // L):
            s = pl.ds(k * L, L)
            norm_v[s] = (tab_v[s] - mean_v[s]) / scale_v[s]

        lane = lax.iota(jnp.int32, L)
        qr = []
        for v in range(F):
            j = lane + v * L
            q = lax.shift_right_logical(j * 205, 10)
            r = j - q * 5
            qr.append((q, r))

        def chunk_body(g, _):
            off = base + g * chunk
            pltpu.sync_copy(idx_hbm.at[pl.ds(off, chunk)], idx_v)

            def group_body(i, _):
                i16 = i * L
                for v in range(F):
                    q, r = qr[v]
                    ids = plsc.load_gather(idx_v, [q + i16])
                    vals = plsc.load_gather(norm_v, [ids * F + r])
                    out_v[pl.ds(i * (L * F) + v * L, L)] = vals
                return 0

            lax.fori_loop(0, chunk // L, group_body, 0)
            pltpu.sync_copy(out_v, out_hbm.at[pl.ds(off * F, chunk * F)])
            return 0

        lax.fori_loop(0, n_chunks, chunk_body, 0)

    return enc


def kernel(indices, aa_feature_table, mean_tensor, scale_tensor):
    b, l = indices.shape
    n_idx = b * l
    idx_flat = indices.reshape(n_idx).astype(jnp.int32)

    tab_flat = jnp.pad(aa_feature_table.reshape(-1), (0, TPAD - 22 * F))
    mean_flat = jnp.pad(jnp.tile(mean_tensor, 22), (0, TPAD - 22 * F))
    scale_flat = jnp.pad(jnp.tile(scale_tensor, 22), (0, TPAD - 22 * F),
                         constant_values=1.0)

    enc = _encoder_grid(n_idx, chunk=4096)
    out_flat = enc(idx_flat, tab_flat, mean_flat, scale_flat)
    return out_flat.reshape(b, l, F)
